# baseline (speedup 1.0000x reference)
import functools

import jax
import jax.numpy as jnp
from jax import lax
from jax.experimental import pallas as pl
from jax.experimental.pallas import tpu as pltpu
from jax.experimental.pallas import tpu_sc as plsc

BATCH = 4096
HIST = 50
EMBED_DIM = 64
PADDED_DIM = 128
LANES = 16

NUM_CORES = 2
NUM_SUBCORES = 16
NUM_WORKERS = NUM_CORES * NUM_SUBCORES

ROWS_PER_WORKER = BATCH // NUM_WORKERS
NGRP = ROWS_PER_WORKER // LANES
NCHUNK = HIST
NBUF = 2


@jax.jit
def _sc_embedding_lookup(ids_t, table128):
    mesh = plsc.VectorSubcoreMesh(
        core_axis_name="c", subcore_axis_name="s",
        num_cores=NUM_CORES, num_subcores=NUM_SUBCORES)

    @functools.partial(
        pl.kernel,
        out_type=jax.ShapeDtypeStruct((HIST, EMBED_DIM, BATCH), jnp.float32),
        mesh=mesh,
        scratch_types=[
            pltpu.VMEM((HIST, ROWS_PER_WORKER), jnp.int32),
            pltpu.VMEM((NBUF, ROWS_PER_WORKER, PADDED_DIM), jnp.float32),
            pltpu.VMEM((NBUF, EMBED_DIM, ROWS_PER_WORKER), jnp.float32),
            pltpu.SemaphoreType.DMA((NBUF,)),
            pltpu.SemaphoreType.DMA((NBUF,)),
        ],
        compiler_params=pltpu.CompilerParams(needs_layout_passes=False),
    )
    def k(ids_hbm, table_hbm, out_hbm, idx_v, rows_v, trows_v, gsems, wsems):
        wid = lax.axis_index("s") * NUM_CORES + lax.axis_index("c")
        base = wid * ROWS_PER_WORKER
        pltpu.sync_copy(ids_hbm.at[:, pl.ds(base, ROWS_PER_WORKER)], idx_v)

        row_ids = [lax.iota(jnp.int32, LANES) + g * LANES for g in range(NGRP)]

        def transpose_chunk(s):
            rows = rows_v.at[s]
            trows = trows_v.at[s]
            for e in range(EMBED_DIM):
                col = jnp.full((LANES,), e, jnp.int32)
                for g in range(NGRP):
                    v = plsc.load_gather(rows, [row_ids[g], col])
                    trows[e, pl.ds(g * LANES, LANES)] = v

        for s in range(NBUF):
            pltpu.async_copy(table_hbm.at[idx_v.at[s]], rows_v.at[s],
                             gsems.at[s])

        def outer(g, _):
            for s in range(NBUF):
                h = g * NBUF + s
                pltpu.make_async_copy(
                    table_hbm.at[idx_v.at[h]], rows_v.at[s],
                    gsems.at[s]).wait()

                @pl.when(g > 0)
                def _():
                    pltpu.make_async_copy(
                        trows_v.at[s],
                        out_hbm.at[h - NBUF, :, pl.ds(base, ROWS_PER_WORKER)],
                        wsems.at[s]).wait()

                transpose_chunk(s)

                @pl.when(h + NBUF < NCHUNK)
                def _():
                    pltpu.async_copy(table_hbm.at[idx_v.at[h + NBUF]],
                                     rows_v.at[s], gsems.at[s])

                pltpu.async_copy(
                    trows_v.at[s],
                    out_hbm.at[h, :, pl.ds(base, ROWS_PER_WORKER)],
                    wsems.at[s])
            return 0

        lax.fori_loop(0, NCHUNK // NBUF, outer, 0)

        for s in range(NBUF):
            h = NCHUNK - NBUF + s
            pltpu.make_async_copy(
                trows_v.at[s],
                out_hbm.at[h, :, pl.ds(base, ROWS_PER_WORKER)],
                wsems.at[s]).wait()

    return k(ids_t, table128)


def kernel(satellite_ids, embedding_table):
    ids_t = satellite_ids.T
    table128 = jnp.pad(embedding_table, ((0, 0), (0, PADDED_DIM - EMBED_DIM)))
    out_t = _sc_embedding_lookup(ids_t, table128)
    return jnp.transpose(out_t, (2, 0, 1))


# speedup vs baseline: 1.7255x; 1.7255x over previous
import functools

import jax
import jax.numpy as jnp
from jax import lax
from jax.experimental import pallas as pl
from jax.experimental.pallas import tpu as pltpu
from jax.experimental.pallas import tpu_sc as plsc

BATCH = 4096
HIST = 50
EMBED_DIM = 64

NUM_CORES = 2
NUM_SUBCORES = 16
NUM_WORKERS = NUM_CORES * NUM_SUBCORES

ROWS_PER_WORKER = BATCH // NUM_WORKERS
NCHUNK = ROWS_PER_WORKER
NBUF = 8


@jax.jit
def _sc_embedding_lookup(ids, table):
    mesh = plsc.VectorSubcoreMesh(
        core_axis_name="c", subcore_axis_name="s",
        num_cores=NUM_CORES, num_subcores=NUM_SUBCORES)

    @functools.partial(
        pl.kernel,
        out_type=jax.ShapeDtypeStruct((BATCH, HIST, EMBED_DIM), jnp.float32),
        mesh=mesh,
        scratch_types=[
            pltpu.VMEM((ROWS_PER_WORKER, HIST), jnp.int32),
            pltpu.VMEM((NBUF, HIST, EMBED_DIM), jnp.float32),
            pltpu.SemaphoreType.DMA((NBUF,)),
            pltpu.SemaphoreType.DMA((NBUF,)),
        ],
        compiler_params=pltpu.CompilerParams(use_tc_tiling_on_sc=False),
    )
    def k(ids_hbm, table_hbm, out_hbm, idx_v, rows_v, gsems, wsems):
        wid = lax.axis_index("s") * NUM_CORES + lax.axis_index("c")
        base = wid * ROWS_PER_WORKER
        pltpu.sync_copy(ids_hbm.at[pl.ds(base, ROWS_PER_WORKER)], idx_v)

        for b in range(NBUF):
            pltpu.async_copy(table_hbm.at[idx_v.at[b]], rows_v.at[b],
                             gsems.at[b])

        def outer(g, _):
            for b in range(NBUF):
                j = g * NBUF + b
                pltpu.make_async_copy(
                    table_hbm.at[idx_v.at[j]], rows_v.at[b],
                    gsems.at[b]).wait()
                pltpu.async_copy(rows_v.at[b], out_hbm.at[base + j],
                                 wsems.at[b])
            for b in range(NBUF):
                j = g * NBUF + b

                @pl.when(j + NBUF < NCHUNK)
                def _():
                    pltpu.make_async_copy(
                        rows_v.at[b], out_hbm.at[base + j],
                        wsems.at[b]).wait()
                    pltpu.async_copy(table_hbm.at[idx_v.at[j + NBUF]],
                                     rows_v.at[b], gsems.at[b])

            return 0

        lax.fori_loop(0, NCHUNK // NBUF, outer, 0)

        for b in range(NBUF):
            j = NCHUNK - NBUF + b
            pltpu.make_async_copy(rows_v.at[b], out_hbm.at[base + j],
                                  wsems.at[b]).wait()

    return k(ids, table)


def kernel(satellite_ids, embedding_table):
    return _sc_embedding_lookup(satellite_ids, embedding_table)


# speedup vs baseline: 1.9336x; 1.1206x over previous
import functools

import jax
import jax.numpy as jnp
from jax import lax
from jax.experimental import pallas as pl
from jax.experimental.pallas import tpu as pltpu
from jax.experimental.pallas import tpu_sc as plsc

BATCH = 4096
HIST = 50
EMBED_DIM = 64
PADDED_DIM = 128
LANES = 16

NUM_CORES = 2
NUM_SUBCORES = 16
NUM_WORKERS = NUM_CORES * NUM_SUBCORES

ROWS_PER_WORKER = BATCH // NUM_WORKERS
NGRP = ROWS_PER_WORKER // LANES
NEBLK = EMBED_DIM // LANES
NCHUNK = HIST
NBUF = 2


@jax.jit
def _sc_embedding_lookup(ids_t, table128):
    mesh = plsc.VectorSubcoreMesh(
        core_axis_name="c", subcore_axis_name="s",
        num_cores=NUM_CORES, num_subcores=NUM_SUBCORES)

    @functools.partial(
        pl.kernel,
        out_type=jax.ShapeDtypeStruct((HIST, EMBED_DIM, BATCH), jnp.float32),
        mesh=mesh,
        scratch_types=[
            pltpu.VMEM((HIST, ROWS_PER_WORKER), jnp.int32),
            pltpu.VMEM((NBUF, ROWS_PER_WORKER, PADDED_DIM), jnp.float32),
            pltpu.VMEM((NBUF, EMBED_DIM, ROWS_PER_WORKER), jnp.float32),
            pltpu.SemaphoreType.DMA((NBUF,)),
            pltpu.SemaphoreType.DMA((NBUF,)),
        ],
        compiler_params=pltpu.CompilerParams(needs_layout_passes=False),
    )
    def k(ids_hbm, table_hbm, out_hbm, idx_v, rows_v, trows_v, gsems, wsems):
        wid = lax.axis_index("s") * NUM_CORES + lax.axis_index("c")
        base = wid * ROWS_PER_WORKER
        pltpu.sync_copy(ids_hbm.at[:, pl.ds(base, ROWS_PER_WORKER)], idx_v)

        iota = lax.iota(jnp.int32, LANES)
        perms = [lax.bitwise_and(iota + d, LANES - 1) for d in range(LANES)]

        def transpose_chunk(s):
            rows = rows_v.at[s]
            trows = trows_v.at[s]

            def per_grp(g, _):
                brow = iota + g * LANES
                for kblk in range(NEBLK):
                    for d in range(LANES):
                        ecol = perms[d] + kblk * LANES
                        v = plsc.load_gather(rows, [brow, ecol])
                        plsc.store_scatter(trows, [ecol, brow], v)
                return 0

            lax.fori_loop(0, NGRP, per_grp, 0)

        for s in range(NBUF):
            pltpu.async_copy(table_hbm.at[idx_v.at[s]], rows_v.at[s],
                             gsems.at[s])

        def outer(g, _):
            for s in range(NBUF):
                h = g * NBUF + s
                pltpu.make_async_copy(
                    table_hbm.at[idx_v.at[h]], rows_v.at[s],
                    gsems.at[s]).wait()

                @pl.when(g > 0)
                def _():
                    pltpu.make_async_copy(
                        trows_v.at[s],
                        out_hbm.at[h - NBUF, :, pl.ds(base, ROWS_PER_WORKER)],
                        wsems.at[s]).wait()

                transpose_chunk(s)

                @pl.when(h + NBUF < NCHUNK)
                def _():
                    pltpu.async_copy(table_hbm.at[idx_v.at[h + NBUF]],
                                     rows_v.at[s], gsems.at[s])

                pltpu.async_copy(
                    trows_v.at[s],
                    out_hbm.at[h, :, pl.ds(base, ROWS_PER_WORKER)],
                    wsems.at[s])
            return 0

        lax.fori_loop(0, NCHUNK // NBUF, outer, 0)

        for s in range(NBUF):
            h = NCHUNK - NBUF + s
            pltpu.make_async_copy(
                trows_v.at[s],
                out_hbm.at[h, :, pl.ds(base, ROWS_PER_WORKER)],
                wsems.at[s]).wait()

    return k(ids_t, table128)


def kernel(satellite_ids, embedding_table):
    ids_t = satellite_ids.T
    table128 = jnp.pad(embedding_table, ((0, 0), (0, PADDED_DIM - EMBED_DIM)))
    out_t = _sc_embedding_lookup(ids_t, table128)
    return jnp.transpose(out_t, (2, 0, 1))
